# baseline (speedup 1.0000x reference)
import functools

import jax
import jax.numpy as jnp
from jax import lax
from jax.experimental import pallas as pl
from jax.experimental.pallas import tpu as pltpu
from jax.experimental.pallas import tpu_sc as plsc

EMB = 64
SCALE = 8.0

NC = 2
NS = 16
NW = NC * NS
NBUF = 4

VOC_MAIN = 999936


def _make_pack(v_main):
    n_cols = v_main // 128
    percol = n_cols // NW
    extra = n_cols - percol * NW
    mesh = plsc.VectorSubcoreMesh(core_axis_name="c", subcore_axis_name="s")

    @functools.partial(
        pl.kernel,
        mesh=mesh,
        out_type=jax.ShapeDtypeStruct((v_main // 2, 128), jnp.float32),
        compiler_params=pltpu.CompilerParams(
            use_tc_tiling_on_sc=True, needs_layout_passes=False),
        scratch_types=[
            [pltpu.VMEM((64, 128), jnp.float32) for _ in range(2)],
            [pltpu.VMEM((64, 128), jnp.float32) for _ in range(2)],
            [pltpu.SemaphoreType.DMA for _ in range(2)],
            [pltpu.SemaphoreType.DMA for _ in range(2)],
        ],
    )
    def pack_kernel(tt_hbm, out_hbm, slabs, obufs, gsems, ssems):
        wid = lax.axis_index("s") * NC + lax.axis_index("c")
        start = wid * percol + jnp.minimum(wid, extra)
        count = percol + (wid < extra).astype(jnp.int32)

        def rd(t, b, sem):
            return pltpu.make_async_copy(
                tt_hbm.at[:, pl.ds(t * 128, 128)], slabs[b], sem)

        def wr(t, b, sem):
            return pltpu.make_async_copy(
                obufs[b], out_hbm.at[pl.ds(t * 64, 64)], sem)

        rd(start, 0, gsems[0]).start()
        lane = lax.iota(jnp.int32, 16)

        @pl.loop(0, (percol + 2) // 2)
        def step2(g):
            for b in range(2):
                k = g * 2 + b

                @pl.when(k < count)
                def _():
                    t = start + k
                    rd(t, b, gsems[b]).wait()

                    @pl.when(k + 1 < count)
                    def _():
                        @pl.when(k >= 1)
                        def _():
                            wr(t - 1, 1 - b, ssems[1 - b]).wait()
                        rd(t + 1, 1 - b, gsems[1 - b]).start()

                    @plsc.parallel_loop(0, 64, unroll=2)
                    def line(i):
                        for k8 in range(8):
                            col = 2 * i + (1 if k8 >= 4 else 0)
                            e = (k8 % 4) * 16 + lane
                            vals = plsc.load_gather(
                                slabs[b], [e, jnp.full((16,), col, jnp.int32)])
                            obufs[b][i, pl.ds((k8 % 4) * 16 + (k8 // 4) * 64, 16)] = vals

                    wr(t, b, ssems[b]).start()

        last = start + count - 1
        par = (count - 1) % 2
        for bb in range(2):
            tbb = last - (par != bb).astype(jnp.int32)
            wr(tbb, bb, ssems[bb]).wait()

    return pack_kernel


def _make_emb(n_b, n_l, v_main):
    b_per_w = n_b // NW
    n_per_w = b_per_w * n_l
    segs = []
    off = 0
    while off < n_l:
        s = min(128, n_l - off)
        s -= s % 8
        segs.append((off, s))
        off += s
    vpb = n_l * EMB // 16
    n_groups16 = (n_l + 15) // 16
    mesh = plsc.VectorSubcoreMesh(core_axis_name="c", subcore_axis_name="s")

    @functools.partial(
        pl.kernel,
        mesh=mesh,
        out_type=jax.ShapeDtypeStruct((n_b, n_l, EMB), jnp.float32),
        compiler_params=pltpu.CompilerParams(
            use_tc_tiling_on_sc=False, needs_layout_passes=False),
        scratch_types=[
            pltpu.VMEM((n_per_w,), jnp.int32),
            pltpu.VMEM((n_per_w,), jnp.int32),
            pltpu.VMEM((64, EMB), jnp.float32),
            [pltpu.VMEM((n_l, EMB), jnp.float32) for _ in range(NBUF)],
            [pltpu.SemaphoreType.DMA for _ in range(NBUF)],
            [pltpu.SemaphoreType.DMA for _ in range(NBUF)],
        ],
    )
    def emb_kernel(idx_hbm, table_hbm, tail_hbm, out_hbm, idx_v, ci_v, tail_v,
                   bufs, gsems, ssems):
        wid = lax.axis_index("s") * NC + lax.axis_index("c")
        base = wid * b_per_w
        pltpu.sync_copy(idx_hbm.at[wid], idx_v)
        pltpu.sync_copy(tail_hbm, tail_v)

        @plsc.parallel_loop(0, n_per_w // 16, unroll=8)
        def clamp(i):
            sl = pl.ds(i * 16, 16)
            ci_v[sl] = jnp.minimum(idx_v[sl], v_main - 1)

        def gather(j, b, sem):
            for off, s in segs:
                pltpu.async_copy(
                    table_hbm.at[ci_v.at[pl.ds(j * n_l + off, s)]],
                    bufs[b].at[pl.ds(off, s)], sem)

        def gather_wait(j, b, sem):
            for off, s in segs:
                pltpu.make_async_copy(
                    table_hbm.at[ci_v.at[pl.ds(j * n_l + off, s)]],
                    bufs[b].at[pl.ds(off, s)], sem).wait()

        def store(j, b, sem):
            return pltpu.make_async_copy(bufs[b], out_hbm.at[base + j], sem)

        for b in range(NBUF - 1):
            gather(b, b, gsems[b])

        lane = lax.iota(jnp.int32, 16)

        @pl.loop(0, b_per_w // NBUF)
        def group(g):
            for b in range(NBUF):
                j = g * NBUF + b
                gather_wait(j, b, gsems[b])

                @plsc.parallel_loop(0, vpb, unroll=8)
                def scale(i):
                    r = i >> 2
                    sl = pl.ds((i & 3) * 16, 16)
                    bufs[b][r, sl] = bufs[b][r, sl] * SCALE

                def fix_group(g16, _):
                    rows = jnp.minimum(g16 * 16 + lane, n_l - 1)
                    iv = plsc.load_gather(idx_v, [j * n_l + rows])
                    m = (iv >= v_main) & (g16 * 16 + lane < n_l)

                    @pl.when(jnp.any(m))
                    def _():
                        trow = jnp.clip(iv - v_main, 0, 63)

                        def fix_col(c, _):
                            cc = jnp.full((16,), c, jnp.int32)
                            tv = plsc.load_gather(tail_v, [trow, cc])
                            plsc.store_scatter(
                                bufs[b], [rows, cc], tv * SCALE, mask=m)
                            return 0

                        lax.fori_loop(0, EMB, fix_col, 0)
                    return 0

                lax.fori_loop(0, n_groups16, fix_group, 0)

                store(j, b, ssems[b]).start()

                bn = (b + NBUF - 1) % NBUF

                @pl.when(j + NBUF - 1 < b_per_w)
                def _():
                    @pl.when(j >= 1)
                    def _():
                        store(j - 1, bn, ssems[bn]).wait()
                    gather(j + NBUF - 1, bn, gsems[bn])

        for jj in range(b_per_w - NBUF, b_per_w):
            store(jj, jj % NBUF, ssems[jj % NBUF]).wait()

    return emb_kernel


def kernel(x, table):
    n_b, n_l = x.shape
    v, d = table.shape
    packed = _make_pack(VOC_MAIN)(table.T)
    tbl = packed.reshape(VOC_MAIN, d)
    tail = lax.slice(table, (VOC_MAIN, 0), (v, d))
    idx = x.reshape(NW, (n_b // NW) * n_l).astype(jnp.int32)
    return _make_emb(n_b, n_l, VOC_MAIN)(idx, tbl, tail)


# speedup vs baseline: 1.0003x; 1.0003x over previous
import functools

import jax
import jax.numpy as jnp
from jax import lax
from jax.experimental import pallas as pl
from jax.experimental.pallas import tpu as pltpu
from jax.experimental.pallas import tpu_sc as plsc

EMB = 64
SCALE = 8.0

NC = 2
NS = 16
NW = NC * NS
NBUF = 4

VOC_MAIN = 999936


def _make_pack(v_main):
    n_cols = v_main // 128
    percol = n_cols // NW
    extra = n_cols - percol * NW
    mesh = plsc.VectorSubcoreMesh(core_axis_name="c", subcore_axis_name="s")

    @functools.partial(
        pl.kernel,
        mesh=mesh,
        out_type=jax.ShapeDtypeStruct((v_main // 2, 128), jnp.float32),
        compiler_params=pltpu.CompilerParams(
            use_tc_tiling_on_sc=True, needs_layout_passes=False),
        scratch_types=[
            [pltpu.VMEM((64, 128), jnp.float32) for _ in range(2)],
            [pltpu.VMEM((64, 128), jnp.float32) for _ in range(2)],
            [pltpu.SemaphoreType.DMA for _ in range(2)],
            [pltpu.SemaphoreType.DMA for _ in range(2)],
        ],
    )
    def pack_kernel(tt_hbm, out_hbm, slabs, obufs, gsems, ssems):
        wid = lax.axis_index("s") * NC + lax.axis_index("c")
        start = wid * percol + jnp.minimum(wid, extra)
        count = percol + (wid < extra).astype(jnp.int32)

        def rd(t, b, sem):
            return pltpu.make_async_copy(
                tt_hbm.at[:, pl.ds(t * 128, 128)], slabs[b], sem)

        def wr(t, b, sem):
            return pltpu.make_async_copy(
                obufs[b], out_hbm.at[pl.ds(t * 64, 64)], sem)

        rd(start, 0, gsems[0]).start()
        lane = lax.iota(jnp.int32, 16)

        @pl.loop(0, (percol + 2) // 2)
        def step2(g):
            for b in range(2):
                k = g * 2 + b

                @pl.when(k < count)
                def _():
                    t = start + k
                    rd(t, b, gsems[b]).wait()

                    @pl.when(k + 1 < count)
                    def _():
                        @pl.when(k >= 1)
                        def _():
                            wr(t - 1, 1 - b, ssems[1 - b]).wait()
                        rd(t + 1, 1 - b, gsems[1 - b]).start()

                    @plsc.parallel_loop(0, 64, unroll=4)
                    def line(i):
                        c0 = jnp.full((16,), 2 * i, jnp.int32)
                        c1 = c0 + 1
                        for k4 in range(4):
                            e = k4 * 16 + lane
                            v0 = plsc.load_gather(slabs[b], [e, c0])
                            v1 = plsc.load_gather(slabs[b], [e, c1])
                            obufs[b][i, pl.ds(k4 * 16, 16)] = v0
                            obufs[b][i, pl.ds(64 + k4 * 16, 16)] = v1

                    wr(t, b, ssems[b]).start()

        last = start + count - 1
        par = (count - 1) % 2
        for bb in range(2):
            tbb = last - (par != bb).astype(jnp.int32)
            wr(tbb, bb, ssems[bb]).wait()

    return pack_kernel


def _make_emb(n_b, n_l, v_main):
    b_per_w = n_b // NW
    n_per_w = b_per_w * n_l
    segs = []
    off = 0
    while off < n_l:
        s = min(128, n_l - off)
        s -= s % 8
        segs.append((off, s))
        off += s
    vpb = n_l * EMB // 16
    n_groups16 = (n_l + 15) // 16
    mesh = plsc.VectorSubcoreMesh(core_axis_name="c", subcore_axis_name="s")

    @functools.partial(
        pl.kernel,
        mesh=mesh,
        out_type=jax.ShapeDtypeStruct((n_b, n_l, EMB), jnp.float32),
        compiler_params=pltpu.CompilerParams(
            use_tc_tiling_on_sc=False, needs_layout_passes=False),
        scratch_types=[
            pltpu.VMEM((n_per_w,), jnp.int32),
            pltpu.VMEM((n_per_w,), jnp.int32),
            pltpu.VMEM((64, EMB), jnp.float32),
            [pltpu.VMEM((n_l, EMB), jnp.float32) for _ in range(NBUF)],
            [pltpu.SemaphoreType.DMA for _ in range(NBUF)],
            [pltpu.SemaphoreType.DMA for _ in range(NBUF)],
        ],
    )
    def emb_kernel(idx_hbm, table_hbm, tail_hbm, out_hbm, idx_v, ci_v, tail_v,
                   bufs, gsems, ssems):
        wid = lax.axis_index("s") * NC + lax.axis_index("c")
        base = wid * b_per_w
        pltpu.sync_copy(idx_hbm.at[wid], idx_v)
        pltpu.sync_copy(tail_hbm, tail_v)

        @plsc.parallel_loop(0, n_per_w // 16, unroll=8)
        def clamp(i):
            sl = pl.ds(i * 16, 16)
            ci_v[sl] = jnp.minimum(idx_v[sl], v_main - 1)

        def gather(j, b, sem):
            for off, s in segs:
                pltpu.async_copy(
                    table_hbm.at[ci_v.at[pl.ds(j * n_l + off, s)]],
                    bufs[b].at[pl.ds(off, s)], sem)

        def gather_wait(j, b, sem):
            for off, s in segs:
                pltpu.make_async_copy(
                    table_hbm.at[ci_v.at[pl.ds(j * n_l + off, s)]],
                    bufs[b].at[pl.ds(off, s)], sem).wait()

        def store(j, b, sem):
            return pltpu.make_async_copy(bufs[b], out_hbm.at[base + j], sem)

        for b in range(NBUF - 1):
            gather(b, b, gsems[b])

        lane = lax.iota(jnp.int32, 16)

        @pl.loop(0, b_per_w // NBUF)
        def group(g):
            for b in range(NBUF):
                j = g * NBUF + b
                gather_wait(j, b, gsems[b])

                @plsc.parallel_loop(0, vpb, unroll=8)
                def scale(i):
                    r = i >> 2
                    sl = pl.ds((i & 3) * 16, 16)
                    bufs[b][r, sl] = bufs[b][r, sl] * SCALE

                def fix_group(g16, _):
                    rows = jnp.minimum(g16 * 16 + lane, n_l - 1)
                    iv = plsc.load_gather(idx_v, [j * n_l + rows])
                    m = (iv >= v_main) & (g16 * 16 + lane < n_l)

                    @pl.when(jnp.any(m))
                    def _():
                        trow = jnp.clip(iv - v_main, 0, 63)

                        def fix_col(c, _):
                            cc = jnp.full((16,), c, jnp.int32)
                            tv = plsc.load_gather(tail_v, [trow, cc])
                            plsc.store_scatter(
                                bufs[b], [rows, cc], tv * SCALE, mask=m)
                            return 0

                        lax.fori_loop(0, EMB, fix_col, 0)
                    return 0

                lax.fori_loop(0, n_groups16, fix_group, 0)

                store(j, b, ssems[b]).start()

                bn = (b + NBUF - 1) % NBUF

                @pl.when(j + NBUF - 1 < b_per_w)
                def _():
                    @pl.when(j >= 1)
                    def _():
                        store(j - 1, bn, ssems[bn]).wait()
                    gather(j + NBUF - 1, bn, gsems[bn])

        for jj in range(b_per_w - NBUF, b_per_w):
            store(jj, jj % NBUF, ssems[jj % NBUF]).wait()

    return emb_kernel


def kernel(x, table):
    n_b, n_l = x.shape
    v, d = table.shape
    packed = _make_pack(VOC_MAIN)(table.T)
    tbl = packed.reshape(VOC_MAIN, d)
    tail = lax.slice(table, (VOC_MAIN, 0), (v, d))
    idx = x.reshape(NW, (n_b // NW) * n_l).astype(jnp.int32)
    return _make_emb(n_b, n_l, VOC_MAIN)(idx, tbl, tail)


# speedup vs baseline: 1.3666x; 1.3662x over previous
import functools

import jax
import jax.numpy as jnp
from jax import lax
from jax.experimental import pallas as pl
from jax.experimental.pallas import tpu as pltpu
from jax.experimental.pallas import tpu_sc as plsc

EMB = 64
SCALE = 8.0

NC = 2
NS = 16
NW = NC * NS
NBUF = 4
HALF = 2


def _make_emb(n_b, n_l):
    b_per_w = n_b // NW
    segs = []
    off = 0
    while off < n_l:
        s = min(128, n_l - off)
        s -= s % 8
        segs.append((off, s))
        off += s
    vpb = n_l * EMB // 16
    mesh = plsc.VectorSubcoreMesh(core_axis_name="c", subcore_axis_name="s")

    @functools.partial(
        pl.kernel,
        mesh=mesh,
        out_type=jax.ShapeDtypeStruct((n_b, n_l, EMB), jnp.float32),
        compiler_params=pltpu.CompilerParams(use_tc_tiling_on_sc=False),
        scratch_types=[
            pltpu.VMEM((b_per_w, n_l), jnp.int32),
            [pltpu.VMEM((n_l, EMB), jnp.float32) for _ in range(NBUF)],
            [pltpu.SemaphoreType.DMA for _ in range(NBUF)],
            [pltpu.SemaphoreType.DMA for _ in range(NBUF)],
        ],
    )
    def emb_kernel(idx_hbm, table_hbm, out_hbm, idx_v, bufs, gsems, ssems):
        wid = lax.axis_index("s") * NC + lax.axis_index("c")
        base = wid * b_per_w
        pltpu.sync_copy(idx_hbm.at[pl.ds(base, b_per_w)], idx_v)

        def gather(j, b, sem):
            for off, s in segs:
                pltpu.async_copy(
                    table_hbm.at[idx_v.at[j, pl.ds(off, s)]],
                    bufs[b].at[pl.ds(off, s)], sem)

        def gather_wait(j, b, sem):
            for off, s in segs:
                pltpu.make_async_copy(
                    table_hbm.at[idx_v.at[j, pl.ds(off, s)]],
                    bufs[b].at[pl.ds(off, s)], sem).wait()

        def store(j, b, sem):
            return pltpu.make_async_copy(bufs[b], out_hbm.at[base + j], sem)

        for b in range(NBUF - 1):
            gather(b, b, gsems[b])

        @pl.loop(0, b_per_w // NBUF)
        def group(g):
            for b in range(NBUF):
                j = g * NBUF + b
                gather_wait(j, b, gsems[b])

                @plsc.parallel_loop(0, vpb, unroll=8)
                def scale(i):
                    r = i >> 2
                    sl = pl.ds((i & 3) * 16, 16)
                    bufs[b][r, sl] = bufs[b][r, sl] * SCALE

                store(j, b, ssems[b]).start()

                bn = (b + NBUF - 1) % NBUF

                @pl.when(j + NBUF - 1 < b_per_w)
                def _():
                    @pl.when(j >= 1)
                    def _():
                        store(j - 1, bn, ssems[bn]).wait()
                    gather(j + NBUF - 1, bn, gsems[bn])

        for jj in range(b_per_w - NBUF, b_per_w):
            store(jj, jj % NBUF, ssems[jj % NBUF]).wait()

    return emb_kernel


def kernel(x, table):
    n_b, n_l = x.shape
    return _make_emb(n_b, n_l)(x.astype(jnp.int32), table)
